# baseline (speedup 1.0000x reference)
import jax
import jax.numpy as jnp
from jax.experimental import pallas as pl
from jax.experimental.pallas import tpu as pltpu

B, C_IN, N = 16, 128, 1024
C_OUT = 256
C_CAT = 5 * C_IN
BPS = 4
NBLK = B // BPS


def _gnn_body(x_ref, a_ref, w_ref, b_ref, y_ref, aa_ref, xc_ref):
    @pl.when(pl.program_id(0) == 0)
    def _build_aa():
        a0 = a_ref[0].astype(jnp.bfloat16)
        a1 = a_ref[1].astype(jnp.bfloat16)
        aa_ref[:, 0 * N:1 * N] = a0
        aa_ref[:, 1 * N:2 * N] = jnp.dot(
            a0, a0, preferred_element_type=jnp.float32).astype(jnp.bfloat16)
        aa_ref[:, 2 * N:3 * N] = a1
        aa_ref[:, 3 * N:4 * N] = jnp.dot(
            a1, a1, preferred_element_type=jnp.float32).astype(jnp.bfloat16)

    w16 = w_ref[...].astype(jnp.bfloat16)
    bias = b_ref[...]
    y4 = jnp.dot(w16, xc_ref[...], preferred_element_type=jnp.float32)
    for i in range(BPS):
        y_ref[i] = y4[:, i * N:(i + 1) * N] + bias

    xb = x_ref[...].reshape(BPS * C_IN, N).astype(jnp.bfloat16)
    H = jnp.dot(xb, aa_ref[...], preferred_element_type=jnp.float32
                ).astype(jnp.bfloat16)
    for i in range(BPS):
        s = slice(i * C_IN, (i + 1) * C_IN)
        lane = slice(i * N, (i + 1) * N)
        xc_ref[0:C_IN, lane] = xb[s]
        for k in range(4):
            xc_ref[(k + 1) * C_IN:(k + 2) * C_IN, lane] = H[s, k * N:(k + 1) * N]


def kernel(x, adjs, W, b):
    b2d = b.reshape(C_OUT, 1)
    grid = (NBLK + 1,)
    return pl.pallas_call(
        _gnn_body,
        grid=grid,
        in_specs=[
            pl.BlockSpec((BPS, C_IN, N), lambda i: (jnp.minimum(i, NBLK - 1), 0, 0)),
            pl.BlockSpec((2, N, N), lambda i: (0, 0, 0)),
            pl.BlockSpec((C_OUT, C_CAT), lambda i: (0, 0)),
            pl.BlockSpec((C_OUT, 1), lambda i: (0, 0)),
        ],
        out_specs=pl.BlockSpec(
            (BPS, C_OUT, N), lambda i: (jnp.maximum(i - 1, 0), 0, 0)),
        out_shape=jax.ShapeDtypeStruct((B, C_OUT, N), jnp.float32),
        scratch_shapes=[
            pltpu.VMEM((N, 4 * N), jnp.bfloat16),
            pltpu.VMEM((C_CAT, BPS * N), jnp.bfloat16),
        ],
    )(x, adjs, W, b2d)


# speedup vs baseline: 1.2897x; 1.2897x over previous
import jax
import jax.numpy as jnp
from jax.experimental import pallas as pl

B, C_IN, N = 16, 128, 1024
C_OUT = 256
BPS = 4


def _gnn_body(x_ref, a_ref, w_ref, b_ref, y_ref):
    a0 = a_ref[0].astype(jnp.bfloat16)
    a1 = a_ref[1].astype(jnp.bfloat16)
    xb = x_ref[...].reshape(BPS * C_IN, N).astype(jnp.bfloat16)

    h1 = jnp.dot(xb, a0, preferred_element_type=jnp.float32).astype(jnp.bfloat16)
    h3 = jnp.dot(xb, a1, preferred_element_type=jnp.float32).astype(jnp.bfloat16)
    h2 = jnp.dot(h1, a0, preferred_element_type=jnp.float32).astype(jnp.bfloat16)
    h4 = jnp.dot(h3, a1, preferred_element_type=jnp.float32).astype(jnp.bfloat16)

    rows = [
        jnp.concatenate([p[i * C_IN:(i + 1) * C_IN] for i in range(BPS)], axis=1)
        for p in (xb, h1, h2, h3, h4)
    ]
    xc = jnp.concatenate(rows, axis=0)
    w16 = w_ref[...].astype(jnp.bfloat16)
    bias = b_ref[...]
    y4 = jnp.dot(w16, xc, preferred_element_type=jnp.float32)
    for i in range(BPS):
        y_ref[i] = y4[:, i * N:(i + 1) * N] + bias


def kernel(x, adjs, W, b):
    b2d = b.reshape(C_OUT, 1)
    grid = (B // BPS,)
    return pl.pallas_call(
        _gnn_body,
        grid=grid,
        in_specs=[
            pl.BlockSpec((BPS, C_IN, N), lambda i: (i, 0, 0)),
            pl.BlockSpec((2, N, N), lambda i: (0, 0, 0)),
            pl.BlockSpec((C_OUT, 5 * C_IN), lambda i: (0, 0)),
            pl.BlockSpec((C_OUT, 1), lambda i: (0, 0)),
        ],
        out_specs=pl.BlockSpec((BPS, C_OUT, N), lambda i: (i, 0, 0)),
        out_shape=jax.ShapeDtypeStruct((B, C_OUT, N), jnp.float32),
    )(x, adjs, W, b2d)
